# Initial kernel scaffold; baseline (speedup 1.0000x reference)
import jax
import jax.numpy as jnp
from jax.experimental import pallas as pl


def kernel(encoded_identifiers, cfg_nodes_expressions, cfg_nodes_control_kind, W_expr, b_expr, control_kind_table):
    raise NotImplementedError("write your pallas kernel here")


# speedup vs baseline: 4.7460x; 4.7460x over previous
import functools

import jax
import jax.numpy as jnp
from jax import lax
from jax.experimental import pallas as pl
from jax.experimental.pallas import tpu as pltpu
from jax.experimental.pallas import tpu_sc as plsc

N_NODES = 16384
EXPR_LEN = 50
IDENT_DIM = 128
CONTROL_VOCAB = 32
CONTROL_DIM = 8
OUT_DIM = IDENT_DIM + CONTROL_DIM

NBUF = 2
OUT_CHUNK = 128


def _sc_pool_sum(expr_idx, table):
    info = plsc.get_sparse_core_info()
    nc, ns, nlanes = info.num_cores, info.num_subcores, info.num_lanes
    nw = nc * ns
    npw = N_NODES // nw

    mesh = plsc.VectorSubcoreMesh(core_axis_name="c", subcore_axis_name="s")

    @functools.partial(
        pl.kernel,
        mesh=mesh,
        out_type=jax.ShapeDtypeStruct((N_NODES, IDENT_DIM), jnp.float32),
        scratch_types=[
            pltpu.VMEM((npw, EXPR_LEN), jnp.int32),
            pltpu.VMEM((NBUF, EXPR_LEN, IDENT_DIM), jnp.float32),
            pltpu.VMEM((OUT_CHUNK, IDENT_DIM), jnp.float32),
        ]
        + [pltpu.SemaphoreType.DMA for _ in range(NBUF)],
    )
    def body(idx_hbm, table_hbm, out_hbm, idx_v, ring_v, out_v, *sems):
        wid = lax.axis_index("s") * nc + lax.axis_index("c")
        base = wid * npw
        pltpu.sync_copy(idx_hbm.at[pl.ds(base, npw)], idx_v)

        def start(j, b):
            pltpu.async_copy(table_hbm.at[idx_v.at[j]], ring_v.at[b], sems[b])

        def wait(b):
            pltpu.make_async_copy(
                table_hbm.at[idx_v.at[0]], ring_v.at[b], sems[b]
            ).wait()

        for b in range(NBUF):
            start(b, b)

        for k in range(npw // OUT_CHUNK):
            k0 = k * OUT_CHUNK

            def outer(g, carry):
                j0 = k0 + g * NBUF
                for b in range(NBUF):
                    j = j0 + b
                    wait(b)
                    for c in range(IDENT_DIM // nlanes):
                        sl = pl.ds(c * nlanes, nlanes)
                        acc = ring_v[b, 0, sl]
                        for r in range(1, EXPR_LEN):
                            acc = acc + ring_v[b, r, sl]
                        out_v[j - k0, sl] = acc

                    @pl.when(j + NBUF < npw)
                    def _start_next():
                        start(j + NBUF, b)

                return carry

            lax.fori_loop(0, OUT_CHUNK // NBUF, outer, 0, unroll=False)
            pltpu.sync_copy(out_v, out_hbm.at[pl.ds(base + k0, OUT_CHUNK)])

    return body(expr_idx, table)


def _tc_project(pooled_sum, W_expr, b_expr, control_kind, control_kind_table):
    bn = 2048
    grid = (N_NODES // bn,)

    def body(x_ref, w_ref, b_ref, ck_ref, ctab_ref, o_ref):
        x = x_ref[...] * (1.0 / EXPR_LEN)
        y = jnp.dot(x, w_ref[...], preferred_element_type=jnp.float32)
        y = y + b_ref[...]
        ck = ck_ref[...]
        onehot = (
            ck == lax.broadcasted_iota(jnp.int32, (bn, CONTROL_VOCAB), 1)
        ).astype(jnp.float32)
        ctl = jnp.dot(onehot, ctab_ref[...], preferred_element_type=jnp.float32)
        o_ref[...] = jnp.concatenate([y, ctl], axis=-1)

    return pl.pallas_call(
        body,
        grid=grid,
        in_specs=[
            pl.BlockSpec((bn, IDENT_DIM), lambda i: (i, 0)),
            pl.BlockSpec((IDENT_DIM, IDENT_DIM), lambda i: (0, 0)),
            pl.BlockSpec((1, IDENT_DIM), lambda i: (0, 0)),
            pl.BlockSpec((bn, 1), lambda i: (i, 0)),
            pl.BlockSpec((CONTROL_VOCAB, CONTROL_DIM), lambda i: (0, 0)),
        ],
        out_specs=pl.BlockSpec((bn, OUT_DIM), lambda i: (i, 0)),
        out_shape=jax.ShapeDtypeStruct((N_NODES, OUT_DIM), jnp.float32),
    )(pooled_sum, W_expr, b_expr, control_kind, control_kind_table)


def kernel(encoded_identifiers, cfg_nodes_expressions, cfg_nodes_control_kind,
           W_expr, b_expr, control_kind_table):
    expr_idx = cfg_nodes_expressions.astype(jnp.int32)
    ck = cfg_nodes_control_kind.astype(jnp.int32).reshape(N_NODES, 1)
    pooled_sum = _sc_pool_sum(expr_idx, encoded_identifiers)
    return _tc_project(
        pooled_sum,
        W_expr,
        b_expr.reshape(1, IDENT_DIM),
        ck,
        control_kind_table,
    )


# speedup vs baseline: 5.8175x; 1.2258x over previous
import functools

import numpy as np

import jax
import jax.numpy as jnp
from jax import lax
from jax.experimental import pallas as pl
from jax.experimental.pallas import tpu as pltpu
from jax.experimental.pallas import tpu_sc as plsc

N_NODES = 16384
EXPR_LEN = 50
IDENT_DIM = 128
CONTROL_VOCAB = 32
CONTROL_DIM = 8
OUT_DIM = IDENT_DIM + CONTROL_DIM

NBUF = 4
PAIR = 2
FLUSH_GROUPS = 16

_PERM = np.empty((IDENT_DIM,), dtype=np.int32)
for _c in range(IDENT_DIM // 32):
    for _k in range(16):
        _PERM[32 * _c + _k] = 32 * _c + 2 * _k
        _PERM[32 * _c + 16 + _k] = 32 * _c + 2 * _k + 1


def _sc_pool_sum(pair_idx, packed_table):
    info = plsc.get_sparse_core_info()
    nc, ns, nlanes = info.num_cores, info.num_subcores, info.num_lanes
    nw = nc * ns
    npw = N_NODES // nw
    ppw = npw // PAIR
    nchunks = IDENT_DIM // 32
    rows = PAIR * EXPR_LEN
    out_rows = FLUSH_GROUPS * NBUF * PAIR

    mesh = plsc.VectorSubcoreMesh(core_axis_name="c", subcore_axis_name="s")

    @functools.partial(
        pl.kernel,
        mesh=mesh,
        compiler_params=pltpu.CompilerParams(
            needs_layout_passes=False, use_tc_tiling_on_sc=False),
        out_type=jax.ShapeDtypeStruct((N_NODES, IDENT_DIM), jnp.float32),
        scratch_types=[
            pltpu.VMEM((ppw, rows), jnp.int32),
            pltpu.VMEM((NBUF, rows, IDENT_DIM // 2), jnp.int32),
            pltpu.VMEM((out_rows, IDENT_DIM), jnp.float32),
        ]
        + [pltpu.SemaphoreType.DMA for _ in range(NBUF)],
    )
    def body(idx_hbm, table_hbm, out_hbm, idx_v, ring_v, out_v, *sems):
        wid = lax.axis_index("s") * nc + lax.axis_index("c")
        pbase = wid * ppw
        nbase = wid * npw
        pltpu.sync_copy(idx_hbm.at[pl.ds(pbase, ppw)], idx_v)

        def start(p, b):
            pltpu.async_copy(table_hbm.at[idx_v.at[p]], ring_v.at[b], sems[b])

        def wait(b):
            pltpu.make_async_copy(
                table_hbm.at[idx_v.at[0]], ring_v.at[b], sems[b]
            ).wait()

        for b in range(NBUF):
            start(b, b)

        def outer(g, carry):
            p0 = g * NBUF
            kk = g // FLUSH_GROUPS
            for b in range(NBUF):
                p = p0 + b
                wait(b)
                for half in range(PAIR):
                    row0 = half * EXPR_LEN
                    orow = (p - kk * (FLUSH_GROUPS * NBUF)) * PAIR + half

                    hi_mask = jnp.int32(-65536)

                    def rbody(r, accs):
                        out = []
                        for c in range(nchunks):
                            w = ring_v[b, row0 + r, pl.ds(16 * c, 16)]
                            ev = lax.bitcast_convert_type(
                                lax.shift_left(w, 16), jnp.float32)
                            od = lax.bitcast_convert_type(
                                w & hi_mask, jnp.float32)
                            out.append(accs[2 * c] + ev)
                            out.append(accs[2 * c + 1] + od)
                        return tuple(out)

                    zero = jnp.zeros((nlanes,), jnp.float32)
                    accs = lax.fori_loop(
                        0, EXPR_LEN, rbody, (zero,) * (2 * nchunks),
                        unroll=10)
                    for c in range(nchunks):
                        out_v[orow, pl.ds(32 * c, nlanes)] = accs[2 * c]
                        out_v[orow, pl.ds(32 * c + 16, nlanes)] = accs[2 * c + 1]

                    @pl.when((p + NBUF < ppw) & (half == PAIR - 1))
                    def _start_next():
                        start(p + NBUF, b)

            @pl.when(g % FLUSH_GROUPS == FLUSH_GROUPS - 1)
            def _flush():
                pltpu.sync_copy(
                    out_v, out_hbm.at[pl.ds(nbase + kk * out_rows, out_rows)])

            return carry

        lax.fori_loop(0, ppw // NBUF, outer, 0, unroll=False)

    return body(pair_idx, packed_table)


def _tc_project(pooled_sum, W_perm, b_expr, control_kind, control_kind_table):
    bn = 2048
    grid = (N_NODES // bn,)

    def body(x_ref, w_ref, b_ref, ck_ref, ctab_ref, o_ref):
        x = x_ref[...] * (1.0 / EXPR_LEN)
        y = jnp.dot(x, w_ref[...], preferred_element_type=jnp.float32)
        y = y + b_ref[...]
        ck = ck_ref[...]
        onehot = (
            ck == lax.broadcasted_iota(jnp.int32, (bn, CONTROL_VOCAB), 1)
        ).astype(jnp.float32)
        ctl = jnp.dot(onehot, ctab_ref[...], preferred_element_type=jnp.float32)
        o_ref[...] = jnp.concatenate([y, ctl], axis=-1)

    return pl.pallas_call(
        body,
        grid=grid,
        in_specs=[
            pl.BlockSpec((bn, IDENT_DIM), lambda i: (i, 0)),
            pl.BlockSpec((IDENT_DIM, IDENT_DIM), lambda i: (0, 0)),
            pl.BlockSpec((1, IDENT_DIM), lambda i: (0, 0)),
            pl.BlockSpec((bn, 1), lambda i: (i, 0)),
            pl.BlockSpec((CONTROL_VOCAB, CONTROL_DIM), lambda i: (0, 0)),
        ],
        out_specs=pl.BlockSpec((bn, OUT_DIM), lambda i: (i, 0)),
        out_shape=jax.ShapeDtypeStruct((N_NODES, OUT_DIM), jnp.float32),
    )(pooled_sum, W_perm, b_expr, control_kind, control_kind_table)


def kernel(encoded_identifiers, cfg_nodes_expressions, cfg_nodes_control_kind,
           W_expr, b_expr, control_kind_table):
    table_bf = encoded_identifiers.astype(jnp.bfloat16)
    packed = lax.bitcast_convert_type(
        table_bf.reshape(-1, IDENT_DIM // 2, 2), jnp.int32)
    pair_idx = cfg_nodes_expressions.astype(jnp.int32).reshape(
        N_NODES // PAIR, PAIR * EXPR_LEN)
    ck = cfg_nodes_control_kind.astype(jnp.int32).reshape(N_NODES, 1)
    W_perm = W_expr[_PERM, :]

    pooled_sum = _sc_pool_sum(pair_idx, packed)
    return _tc_project(
        pooled_sum,
        W_perm,
        b_expr.reshape(1, IDENT_DIM),
        ck,
        control_kind_table,
    )


# speedup vs baseline: 12.2806x; 2.1110x over previous
import functools

import jax
import jax.numpy as jnp
from jax import lax
from jax.experimental import pallas as pl
from jax.experimental.pallas import tpu as pltpu
from jax.experimental.pallas import tpu_sc as plsc

N_NODES = 16384
EXPR_LEN = 50
IDENT_DIM = 128
HALF = IDENT_DIM // 2
CONTROL_VOCAB = 32
CONTROL_DIM = 8
OUT_DIM = IDENT_DIM + CONTROL_DIM
N_IDENT = 100000

GROUP = 4
NBUF = 4
FLUSH_GROUPS = 32
PACK_CHUNK = 125


def _sc_mesh_info():
    info = plsc.get_sparse_core_info()
    return info.num_cores, info.num_subcores, info.num_lanes


def _sc_pack_table(table_f32):
    nc, ns, nlanes = _sc_mesh_info()
    nw = nc * ns
    rpw = N_IDENT // nw
    nsteps = rpw // PACK_CHUNK

    mesh = plsc.VectorSubcoreMesh(core_axis_name="c", subcore_axis_name="s")

    @functools.partial(
        pl.kernel,
        mesh=mesh,
        compiler_params=pltpu.CompilerParams(
            needs_layout_passes=False, use_tc_tiling_on_sc=False),
        out_type=jax.ShapeDtypeStruct((N_IDENT, HALF), jnp.int32),
        scratch_types=[
            pltpu.VMEM((PACK_CHUNK, IDENT_DIM), jnp.float32),
            pltpu.VMEM((PACK_CHUNK, HALF), jnp.int32),
        ],
    )
    def body(tab_hbm, out_hbm, in_v, out_v):
        wid = lax.axis_index("s") * nc + lax.axis_index("c")
        base = wid * rpw
        rnd = jnp.int32(0x8000)
        hi_mask = jnp.int32(-65536)
        lo_mask = jnp.int32(0xFFFF)

        def step(s, carry):
            row0 = base + s * PACK_CHUNK
            pltpu.sync_copy(tab_hbm.at[pl.ds(row0, PACK_CHUNK)], in_v)

            def rbody(r, c2):
                for c in range(HALF // 16):
                    a = lax.bitcast_convert_type(
                        in_v[r, pl.ds(16 * c, 16)], jnp.int32)
                    bvec = lax.bitcast_convert_type(
                        in_v[r, pl.ds(HALF + 16 * c, 16)], jnp.int32)
                    lo = lax.shift_right_logical(a + rnd, 16) & lo_mask
                    hi = (bvec + rnd) & hi_mask
                    out_v[r, pl.ds(16 * c, 16)] = lo | hi
                return c2

            lax.fori_loop(0, PACK_CHUNK, rbody, 0, unroll=5)
            pltpu.sync_copy(out_v, out_hbm.at[pl.ds(row0, PACK_CHUNK)])
            return carry

        lax.fori_loop(0, nsteps, step, 0, unroll=False)

    return body(table_f32)


def _sc_pool_sum(flat_idx, packed_table):
    nc, ns, nlanes = _sc_mesh_info()
    nw = nc * ns
    npw = N_NODES // nw
    ngroups = npw // GROUP
    rows = GROUP * EXPR_LEN
    ipw = npw * EXPR_LEN
    out_rows = FLUSH_GROUPS * GROUP
    nchunks = HALF // nlanes

    mesh = plsc.VectorSubcoreMesh(core_axis_name="c", subcore_axis_name="s")

    @functools.partial(
        pl.kernel,
        mesh=mesh,
        compiler_params=pltpu.CompilerParams(
            needs_layout_passes=False, use_tc_tiling_on_sc=False),
        out_type=jax.ShapeDtypeStruct((N_NODES, IDENT_DIM), jnp.float32),
        scratch_types=[
            pltpu.VMEM((ipw,), jnp.int32),
            pltpu.VMEM((NBUF, rows, HALF), jnp.int32),
            pltpu.VMEM((out_rows, IDENT_DIM), jnp.float32),
        ]
        + [pltpu.SemaphoreType.DMA for _ in range(NBUF)],
    )
    def body(idx_hbm, table_hbm, out_hbm, idx_v, ring_v, out_v, *sems):
        wid = lax.axis_index("s") * nc + lax.axis_index("c")
        nbase = wid * npw
        pltpu.sync_copy(idx_hbm.at[pl.ds(wid * ipw, ipw)], idx_v)

        def start(g, b):
            off = g * rows
            pltpu.async_copy(
                table_hbm.at[idx_v.at[pl.ds(off, 128)]],
                ring_v.at[b, pl.ds(0, 128)], sems[b])
            pltpu.async_copy(
                table_hbm.at[idx_v.at[pl.ds(off + 128, rows - 128)]],
                ring_v.at[b, pl.ds(128, rows - 128)], sems[b])

        def wait(b):
            pltpu.make_async_copy(
                table_hbm.at[idx_v.at[pl.ds(0, 128)]],
                ring_v.at[b, pl.ds(0, 128)], sems[b]).wait()
            pltpu.make_async_copy(
                table_hbm.at[idx_v.at[pl.ds(0, rows - 128)]],
                ring_v.at[b, pl.ds(128, rows - 128)], sems[b]).wait()

        for b in range(NBUF):
            start(b, b)

        hi_mask = jnp.int32(-65536)

        def outer(gg, carry):
            g0 = gg * NBUF
            kk = gg // (FLUSH_GROUPS // NBUF)
            for b in range(NBUF):
                g = g0 + b
                wait(b)
                for h in range(GROUP):
                    row0 = h * EXPR_LEN
                    orow = (g - kk * FLUSH_GROUPS) * GROUP + h

                    def rbody(r, accs):
                        out = []
                        for c in range(nchunks):
                            w = ring_v[b, row0 + r, pl.ds(16 * c, 16)]
                            lo = lax.bitcast_convert_type(
                                lax.shift_left(w, 16), jnp.float32)
                            hi = lax.bitcast_convert_type(
                                w & hi_mask, jnp.float32)
                            out.append(accs[2 * c] + lo)
                            out.append(accs[2 * c + 1] + hi)
                        return tuple(out)

                    zero = jnp.zeros((nlanes,), jnp.float32)
                    accs = lax.fori_loop(
                        0, EXPR_LEN, rbody, (zero,) * (2 * nchunks),
                        unroll=10)
                    for c in range(nchunks):
                        out_v[orow, pl.ds(16 * c, 16)] = accs[2 * c]
                        out_v[orow, pl.ds(HALF + 16 * c, 16)] = accs[2 * c + 1]

                    if h == GROUP - 1:
                        @pl.when(g + NBUF < ngroups)
                        def _start_next():
                            start(g + NBUF, b)

            @pl.when(gg % (FLUSH_GROUPS // NBUF) == FLUSH_GROUPS // NBUF - 1)
            def _flush():
                pltpu.sync_copy(
                    out_v,
                    out_hbm.at[pl.ds(nbase + kk * out_rows, out_rows)])

            return carry

        lax.fori_loop(0, ngroups // NBUF, outer, 0, unroll=False)

    return body(flat_idx, packed_table)


def _tc_project(pooled_sum, W_expr, b_expr, control_kind, control_kind_table):
    bn = 2048
    grid = (N_NODES // bn,)

    def body(x_ref, w_ref, b_ref, ck_ref, ctab_ref, o_ref):
        x = x_ref[...] * (1.0 / EXPR_LEN)
        y = jnp.dot(x, w_ref[...], preferred_element_type=jnp.float32)
        y = y + b_ref[...]
        ck = ck_ref[...]
        onehot = (
            ck == lax.broadcasted_iota(jnp.int32, (bn, CONTROL_VOCAB), 1)
        ).astype(jnp.float32)
        ctl = jnp.dot(onehot, ctab_ref[...], preferred_element_type=jnp.float32)
        o_ref[...] = jnp.concatenate([y, ctl], axis=-1)

    return pl.pallas_call(
        body,
        grid=grid,
        in_specs=[
            pl.BlockSpec((bn, IDENT_DIM), lambda i: (i, 0)),
            pl.BlockSpec((IDENT_DIM, IDENT_DIM), lambda i: (0, 0)),
            pl.BlockSpec((1, IDENT_DIM), lambda i: (0, 0)),
            pl.BlockSpec((bn, 1), lambda i: (i, 0)),
            pl.BlockSpec((CONTROL_VOCAB, CONTROL_DIM), lambda i: (0, 0)),
        ],
        out_specs=pl.BlockSpec((bn, OUT_DIM), lambda i: (i, 0)),
        out_shape=jax.ShapeDtypeStruct((N_NODES, OUT_DIM), jnp.float32),
    )(pooled_sum, W_expr, b_expr, control_kind, control_kind_table)


def kernel(encoded_identifiers, cfg_nodes_expressions, cfg_nodes_control_kind,
           W_expr, b_expr, control_kind_table):
    flat_idx = cfg_nodes_expressions.astype(jnp.int32).reshape(-1)
    ck = cfg_nodes_control_kind.astype(jnp.int32).reshape(N_NODES, 1)

    packed = _sc_pack_table(encoded_identifiers)
    pooled_sum = _sc_pool_sum(flat_idx, packed)
    return _tc_project(
        pooled_sum,
        W_expr,
        b_expr.reshape(1, IDENT_DIM),
        ck,
        control_kind_table,
    )
